# baseline (speedup 1.0000x reference)
import functools

import jax
import jax.numpy as jnp
from jax import lax
from jax.experimental import pallas as pl
from jax.experimental.pallas import tpu as pltpu
from jax.experimental.pallas import tpu_sc as plsc

N = 10000
E = 160000
FRAMES = 256
HID = 1024
OUT = 1024
NOUT = 256

LANES = 16
NUM_CORES = 2
NUM_SUBCORES = 16
BATCH = 80
NB = 128
EPTT = BATCH * NB
E_PAD = EPTT * NUM_SUBCORES
NP = 10240
ROWS_PT = NP // NUM_SUBCORES


def _make_segsum(num_chunks):
    chunks_per_core = num_chunks // NUM_CORES
    mesh = plsc.VectorSubcoreMesh(core_axis_name="c", subcore_axis_name="s")

    def body(table, src_p, dst_p, zeros_h, out, *rest):
        src_t, dst_t, s0, s1, d0, d1, r0, r1, acc, g0, g1, zsem = rest
        core = lax.axis_index("c")
        sid = lax.axis_index("s")

        pltpu.sync_copy(src_p.at[pl.ds(sid * EPTT, EPTT)], src_t)
        pltpu.sync_copy(dst_p.at[pl.ds(sid * EPTT, EPTT)], dst_t)

        for ch in range(chunks_per_core):
            chunk = core * chunks_per_core + ch
            off = chunk * N

            def build(b, sidx, didx):
                base = b * BATCH
                for j in range(BATCH // LANES):
                    sl = pl.ds(base + j * LANES, LANES)
                    sidx[pl.ds(j * LANES, LANES)] = src_t[sl] + off
                    didx[pl.ds(j * LANES, LANES)] = dst_t[sl]

            pltpu.async_copy(zeros_h, acc.at[pl.ds(sid * ROWS_PT, ROWS_PT)], zsem)

            build(0, s0, d0)
            pltpu.async_copy(table.at[s0], r0, g0)
            build(1, s1, d1)
            pltpu.async_copy(table.at[s1], r1, g1)
            pltpu.make_async_copy(zeros_h, acc.at[pl.ds(sid * ROWS_PT, ROWS_PT)],
                                  zsem).wait()
            plsc.subcore_barrier()

            def ebody(i, carry):
                pltpu.make_async_copy(table.at[s0], r0, g0).wait()
                pltpu.sync_copy(r0, acc.at[d0], add=True)
                build(2 * i + 2, s0, d0)
                pltpu.async_copy(table.at[s0], r0, g0)
                pltpu.make_async_copy(table.at[s1], r1, g1).wait()
                pltpu.sync_copy(r1, acc.at[d1], add=True)
                build(2 * i + 3, s1, d1)
                pltpu.async_copy(table.at[s1], r1, g1)
                return carry

            lax.fori_loop(0, (NB - 2) // 2, ebody, 0)
            pltpu.make_async_copy(table.at[s0], r0, g0).wait()
            pltpu.sync_copy(r0, acc.at[d0], add=True)
            pltpu.make_async_copy(table.at[s1], r1, g1).wait()
            pltpu.sync_copy(r1, acc.at[d1], add=True)

            plsc.subcore_barrier()
            pltpu.sync_copy(acc.at[pl.ds(sid * ROWS_PT, ROWS_PT)],
                            out.at[pl.ds(chunk * NP + sid * ROWS_PT, ROWS_PT)])

    return pl.kernel(
        body,
        out_type=jax.ShapeDtypeStruct((num_chunks * NP, 128), jnp.float32),
        mesh=mesh,
        scratch_types=(
            [pltpu.VMEM((EPTT,), jnp.int32)] * 2
            + [pltpu.VMEM((BATCH,), jnp.int32)] * 4
            + [pltpu.VMEM((BATCH, 128), jnp.float32)] * 2
            + [pltpu.VMEM_SHARED((NP, 128), jnp.float32)]
            + [pltpu.SemaphoreType.DMA] * 3
        ),
    )


_BN = 2000
_NI = N // _BN


def _mm1_body(a_ref, w_ref, b_ref, o_ref, acc_ref, *, nk):
    k = pl.program_id(2)

    @pl.when(k == 0)
    def _():
        acc_ref[...] = jnp.zeros_like(acc_ref)

    acc_ref[...] += jnp.dot(a_ref[0], w_ref[...],
                            preferred_element_type=jnp.float32)

    @pl.when(k == nk - 1)
    def _():
        o_ref[0] = jnp.maximum(acc_ref[...] + b_ref[...], 0.0)


def _mm_relu_chunked(aggc, W, b):
    ck = aggc.shape[0]
    cout = W.shape[1] // 128
    return pl.pallas_call(
        functools.partial(_mm1_body, nk=ck),
        grid=(_NI, cout, ck),
        in_specs=[
            pl.BlockSpec((1, _BN, 128), lambda i, j, k: (k, i, 0)),
            pl.BlockSpec((128, 128), lambda i, j, k: (k, j)),
            pl.BlockSpec((1, 128), lambda i, j, k: (0, j)),
        ],
        out_specs=pl.BlockSpec((1, _BN, 128), lambda i, j, k: (j, i, 0)),
        out_shape=jax.ShapeDtypeStruct((cout, N, 128), jnp.float32),
        scratch_shapes=[pltpu.VMEM((_BN, 128), jnp.float32)],
    )(aggc, W, b)


def _mm2_body(a_ref, w2_ref, b2_ref, wfc_ref, bfc_ref, o_ref, acc_ref, cs_ref,
              *, nk):
    i = pl.program_id(0)
    k = pl.program_id(1)

    @pl.when(k == 0)
    def _():
        acc_ref[...] = jnp.zeros_like(acc_ref)

    acc_ref[...] += jnp.dot(a_ref[0], w2_ref[...],
                            preferred_element_type=jnp.float32)

    @pl.when(k == nk - 1)
    def _():
        h2 = jnp.maximum(acc_ref[...] + b2_ref[...], 0.0)
        part = jnp.sum(h2, axis=0, keepdims=True)

        @pl.when(i == 0)
        def _():
            cs_ref[...] = part

        @pl.when(i > 0)
        def _():
            cs_ref[...] += part

        @pl.when(i == _NI - 1)
        def _():
            o_ref[...] = (jnp.dot(cs_ref[...] * (1.0 / N), wfc_ref[...],
                                  preferred_element_type=jnp.float32)
                          + bfc_ref[...])


def _final(agg2c, W2, b2, Wfc, bfc):
    ck = agg2c.shape[0]
    return pl.pallas_call(
        functools.partial(_mm2_body, nk=ck),
        grid=(_NI, ck),
        in_specs=[
            pl.BlockSpec((1, _BN, 128), lambda i, k: (k, i, 0)),
            pl.BlockSpec((128, OUT), lambda i, k: (k, 0)),
            pl.BlockSpec((1, OUT), lambda i, k: (0, 0)),
            pl.BlockSpec((OUT, NOUT), lambda i, k: (0, 0)),
            pl.BlockSpec((1, NOUT), lambda i, k: (0, 0)),
        ],
        out_specs=pl.BlockSpec((1, NOUT), lambda i, k: (0, 0)),
        out_shape=jax.ShapeDtypeStruct((1, NOUT), jnp.float32),
        scratch_shapes=[
            pltpu.VMEM((_BN, OUT), jnp.float32),
            pltpu.VMEM((1, OUT), jnp.float32),
        ],
    )(agg2c, W2, b2, Wfc, bfc)


def kernel(node_feats, edge_index, W1, b1, W2, b2, Wfc, bfc):
    src = edge_index[0].astype(jnp.int32)
    dst = edge_index[1].astype(jnp.int32)
    src_p = jnp.concatenate([src, jnp.zeros((E_PAD - E,), jnp.int32)])
    dst_p = jnp.concatenate([dst, jnp.full((E_PAD - E,), N, jnp.int32)])
    zeros_h = jnp.zeros((ROWS_PT, 128), jnp.float32)
    nchunk_in = FRAMES // 128
    nchunk_h = HID // 128

    xc = (node_feats.reshape(N, nchunk_in, 128)
          .transpose(1, 0, 2)
          .reshape(nchunk_in * N, 128))
    agg1 = _make_segsum(nchunk_in)(xc, src_p, dst_p, zeros_h)
    hc = _mm_relu_chunked(agg1.reshape(nchunk_in, NP, 128), W1, b1.reshape(1, HID))
    agg2 = _make_segsum(nchunk_h)(hc.reshape(nchunk_h * N, 128), src_p, dst_p,
                                  zeros_h)
    return _final(agg2.reshape(nchunk_h, NP, 128), W2, b2.reshape(1, OUT),
                  Wfc, bfc.reshape(1, NOUT))


# speedup vs baseline: 1.6427x; 1.6427x over previous
import functools

import jax
import jax.numpy as jnp
from jax import lax
from jax.experimental import pallas as pl
from jax.experimental.pallas import tpu as pltpu
from jax.experimental.pallas import tpu_sc as plsc

N = 10000
E = 160000
FRAMES = 256
HID = 1024
OUT = 1024
NOUT = 256

LANES = 16
NUM_CORES = 2
NUM_SUBCORES = 16
EPT = E // NUM_SUBCORES
BATCH = 80
NB = EPT // BATCH
NP = 10240
ROWS_PT = NP // NUM_SUBCORES
ZROWS = 40


def _make_segsum(num_chunks):
    chunks_per_core = num_chunks // NUM_CORES
    mesh = plsc.VectorSubcoreMesh(core_axis_name="c", subcore_axis_name="s")

    def body(table, src, dst, out, *rest):
        src_t, dst_t, s0, s1, d0, d1, r0, r1, zbuf, acc, g0, g1 = rest
        core = lax.axis_index("c")
        sid = lax.axis_index("s")

        def zinit(i, carry):
            zbuf[i // 8, pl.ds((i % 8) * 16, 16)] = jnp.zeros((16,), jnp.float32)
            return carry

        lax.fori_loop(0, ZROWS * 8, zinit, 0)

        pltpu.sync_copy(src.at[pl.ds(sid * EPT, EPT)], src_t.at[pl.ds(0, EPT)])
        pltpu.sync_copy(dst.at[pl.ds(sid * EPT, EPT)], dst_t.at[pl.ds(0, EPT)])
        for j in range(BATCH // LANES):
            src_t[pl.ds(EPT + j * LANES, LANES)] = jnp.zeros((LANES,), jnp.int32)
            dst_t[pl.ds(EPT + j * LANES, LANES)] = jnp.zeros((LANES,), jnp.int32)

        for ch in range(chunks_per_core):
            chunk = core * chunks_per_core + ch
            off = chunk * N

            def build(b, sidx, didx):
                base = b * BATCH
                for j in range(BATCH // LANES):
                    sl = pl.ds(base + j * LANES, LANES)
                    sidx[pl.ds(j * LANES, LANES)] = src_t[sl] + off
                    didx[pl.ds(j * LANES, LANES)] = dst_t[sl]

            def zcopy(j, carry):
                pltpu.sync_copy(zbuf, acc.at[pl.ds(sid * ROWS_PT + j * ZROWS, ZROWS)])
                return carry

            lax.fori_loop(0, ROWS_PT // ZROWS, zcopy, 0)
            plsc.subcore_barrier()

            build(0, s0, d0)
            pltpu.async_copy(table.at[s0], r0, g0)
            build(1, s1, d1)
            pltpu.async_copy(table.at[s1], r1, g1)

            def ebody(i, carry):
                pltpu.make_async_copy(table.at[s0], r0, g0).wait()
                pltpu.sync_copy(r0, acc.at[d0], add=True)
                build(2 * i + 2, s0, d0)
                pltpu.async_copy(table.at[s0], r0, g0)
                pltpu.make_async_copy(table.at[s1], r1, g1).wait()
                pltpu.sync_copy(r1, acc.at[d1], add=True)
                build(2 * i + 3, s1, d1)
                pltpu.async_copy(table.at[s1], r1, g1)
                return carry

            lax.fori_loop(0, (NB - 1) // 2, ebody, 0)
            pltpu.make_async_copy(table.at[s0], r0, g0).wait()
            pltpu.sync_copy(r0, acc.at[d0], add=True)
            pltpu.make_async_copy(table.at[s1], r1, g1).wait()
            plsc.subcore_barrier()

            pltpu.sync_copy(acc.at[pl.ds(sid * ROWS_PT, ROWS_PT)],
                            out.at[pl.ds(chunk * NP + sid * ROWS_PT, ROWS_PT)])

    return pl.kernel(
        body,
        out_type=jax.ShapeDtypeStruct((num_chunks * NP, 128), jnp.float32),
        mesh=mesh,
        scratch_types=(
            [pltpu.VMEM((EPT + BATCH,), jnp.int32)] * 2
            + [pltpu.VMEM((BATCH,), jnp.int32)] * 4
            + [pltpu.VMEM((BATCH, 128), jnp.float32)] * 2
            + [pltpu.VMEM((ZROWS, 128), jnp.float32),
               pltpu.VMEM_SHARED((NP, 128), jnp.float32)]
            + [pltpu.SemaphoreType.DMA] * 2
        ),
    )


_BN = 2000
_NI = N // _BN


def _mm1_body(a_ref, w_ref, b_ref, o_ref, acc_ref, *, nk):
    k = pl.program_id(2)

    @pl.when(k == 0)
    def _():
        acc_ref[...] = jnp.zeros_like(acc_ref)

    acc_ref[...] += jnp.dot(a_ref[0], w_ref[...],
                            preferred_element_type=jnp.float32)

    @pl.when(k == nk - 1)
    def _():
        o_ref[0] = jnp.maximum(acc_ref[...] + b_ref[...], 0.0)


def _mm_relu_chunked(aggc, W, b):
    ck = aggc.shape[0]
    cout = W.shape[1] // 128
    return pl.pallas_call(
        functools.partial(_mm1_body, nk=ck),
        grid=(_NI, cout, ck),
        in_specs=[
            pl.BlockSpec((1, _BN, 128), lambda i, j, k: (k, i, 0)),
            pl.BlockSpec((128, 128), lambda i, j, k: (k, j)),
            pl.BlockSpec((1, 128), lambda i, j, k: (0, j)),
        ],
        out_specs=pl.BlockSpec((1, _BN, 128), lambda i, j, k: (j, i, 0)),
        out_shape=jax.ShapeDtypeStruct((cout, N, 128), jnp.float32),
        scratch_shapes=[pltpu.VMEM((_BN, 128), jnp.float32)],
    )(aggc, W, b)


def _mm2_body(a_ref, w2_ref, b2_ref, wfc_ref, bfc_ref, o_ref, acc_ref, cs_ref,
              *, nk):
    i = pl.program_id(0)
    k = pl.program_id(1)

    @pl.when(k == 0)
    def _():
        acc_ref[...] = jnp.zeros_like(acc_ref)

    acc_ref[...] += jnp.dot(a_ref[0], w2_ref[...],
                            preferred_element_type=jnp.float32)

    @pl.when(k == nk - 1)
    def _():
        h2 = jnp.maximum(acc_ref[...] + b2_ref[...], 0.0)
        part = jnp.sum(h2, axis=0, keepdims=True)

        @pl.when(i == 0)
        def _():
            cs_ref[...] = part

        @pl.when(i > 0)
        def _():
            cs_ref[...] += part

        @pl.when(i == _NI - 1)
        def _():
            o_ref[...] = (jnp.dot(cs_ref[...] * (1.0 / N), wfc_ref[...],
                                  preferred_element_type=jnp.float32)
                          + bfc_ref[...])


def _final(agg2c, W2, b2, Wfc, bfc):
    ck = agg2c.shape[0]
    return pl.pallas_call(
        functools.partial(_mm2_body, nk=ck),
        grid=(_NI, ck),
        in_specs=[
            pl.BlockSpec((1, _BN, 128), lambda i, k: (k, i, 0)),
            pl.BlockSpec((128, OUT), lambda i, k: (k, 0)),
            pl.BlockSpec((1, OUT), lambda i, k: (0, 0)),
            pl.BlockSpec((OUT, NOUT), lambda i, k: (0, 0)),
            pl.BlockSpec((1, NOUT), lambda i, k: (0, 0)),
        ],
        out_specs=pl.BlockSpec((1, NOUT), lambda i, k: (0, 0)),
        out_shape=jax.ShapeDtypeStruct((1, NOUT), jnp.float32),
        scratch_shapes=[
            pltpu.VMEM((_BN, OUT), jnp.float32),
            pltpu.VMEM((1, OUT), jnp.float32),
        ],
    )(agg2c, W2, b2, Wfc, bfc)


def kernel(node_feats, edge_index, W1, b1, W2, b2, Wfc, bfc):
    src = edge_index[0].astype(jnp.int32)
    dst = edge_index[1].astype(jnp.int32)
    nchunk_in = FRAMES // 128
    nchunk_h = HID // 128

    xc = (node_feats.reshape(N, nchunk_in, 128)
          .transpose(1, 0, 2)
          .reshape(nchunk_in * N, 128))
    agg1 = _make_segsum(nchunk_in)(xc, src, dst)
    hc = _mm_relu_chunked(agg1.reshape(nchunk_in, NP, 128), W1, b1.reshape(1, HID))
    agg2 = _make_segsum(nchunk_h)(hc.reshape(nchunk_h * N, 128), src, dst)
    return _final(agg2.reshape(nchunk_h, NP, 128), W2, b2.reshape(1, OUT),
                  Wfc, bfc.reshape(1, NOUT))
